# speedup vs baseline: 15.9192x; 15.9192x over previous
import jax
import jax.numpy as jnp
from jax.experimental import pallas as pl

_NUM_EXPERTS = 8
_HIDDEN = 1024
_INTER = 1024
_T = 64
_TOPK = 2


def _moe_kernel(sel_ref, x_ref, gu_ref, dn_ref, out_ref):
    e = pl.program_id(0)
    x = x_ref[...]
    gu = gu_ref[0]
    y = jax.lax.dot_general(
        x, gu, (((1,), (1,)), ((), ())),
        preferred_element_type=jnp.float32)
    gate = y[:, :_INTER]
    up = y[:, _INTER:]
    inter = gate * jax.nn.sigmoid(gate) * up
    dn = dn_ref[0]
    o = jax.lax.dot_general(
        inter, dn, (((1,), (1,)), ((), ())),
        preferred_element_type=jnp.float32)
    sel = sel_ref[...]
    for k in range(_TOPK):
        mk = sel[:, k:k + 1] == e
        cur = out_ref[:, k * _HIDDEN:(k + 1) * _HIDDEN]
        out_ref[:, k * _HIDDEN:(k + 1) * _HIDDEN] = jnp.where(mk, o, cur)


def kernel(hidden_states, selected_experts, gate_up_proj, down_proj):
    out_flat = pl.pallas_call(
        _moe_kernel,
        grid=(_NUM_EXPERTS,),
        in_specs=[
            pl.BlockSpec((_T, _TOPK), lambda e: (0, 0)),
            pl.BlockSpec((_T, _HIDDEN), lambda e: (0, 0)),
            pl.BlockSpec((1, 2 * _INTER, _HIDDEN), lambda e: (e, 0, 0)),
            pl.BlockSpec((1, _HIDDEN, _INTER), lambda e: (e, 0, 0)),
        ],
        out_specs=pl.BlockSpec((_T, _TOPK * _HIDDEN), lambda e: (0, 0)),
        out_shape=jax.ShapeDtypeStruct((_T, _TOPK * _HIDDEN), jnp.float32),
    )(selected_experts, hidden_states, gate_up_proj, down_proj)
    return out_flat.reshape(_T, _TOPK, _HIDDEN)
